# baseline (speedup 1.0000x reference)
import functools

import jax
import jax.numpy as jnp
from jax import lax
from jax.experimental import pallas as pl
from jax.experimental.pallas import tpu as pltpu
from jax.experimental.pallas import tpu_sc as plsc

N_NODES = 10000
N_EDGES = 320000
D_FEAT = 128
D_EDGE = 16
D_GLOBAL = 128

_MESH = plsc.VectorSubcoreMesh(core_axis_name="c", subcore_axis_name="s")

_CH = 39


def _accum_edge(buf, nrows, accs):
    def body(i, accs):
        out = []
        for r in range(8):
            a = accs[r]
            for k in range(8):
                a = a + buf[i, r, pl.ds(k * 16, 16)]
            out.append(a)
        return tuple(out)
    return lax.fori_loop(0, nrows, body, accs, unroll=2)


def _accum_node(buf, nrows, accs):
    def body(i, accs):
        out = list(accs)
        for r in range(8):
            for k in range(8):
                out[k] = out[k] + buf[i, r, pl.ds(k * 16, 16)]
        return tuple(out)
    return lax.fori_loop(0, nrows, body, accs, unroll=2)


@functools.partial(
    pl.kernel,
    out_type=(
        jax.ShapeDtypeStruct((32, 128), jnp.float32),
        jax.ShapeDtypeStruct((32, 128), jnp.float32),
    ),
    mesh=_MESH,
    scratch_types=[
        pltpu.VMEM((_CH + 1, 8, 128), jnp.float32),
        pltpu.VMEM((_CH + 1, 8, 128), jnp.float32),
        pltpu.VMEM((1, 8, 128), jnp.float32),
        pltpu.VMEM((128,), jnp.float32),
        pltpu.SemaphoreType.DMA,
        pltpu.SemaphoreType.DMA,
    ],
)
def _sc_sums(edge_hbm, node_hbm, oe_hbm, on_hbm,
             buf0, buf1, bufx, vout, sem0, sem1):
    c = lax.axis_index("c")
    s = lax.axis_index("s")
    w = c * 16 + s

    t0 = c * 2500 + s * 156 + jnp.minimum(s, 4)
    accs = tuple(jnp.zeros((16,), jnp.float32) for _ in range(8))
    cp0 = pltpu.async_copy(edge_hbm.at[pl.ds(t0, _CH)],
                           buf0.at[pl.ds(0, _CH)], sem0)
    cp1 = pltpu.async_copy(edge_hbm.at[pl.ds(t0 + _CH, _CH)],
                           buf1.at[pl.ds(0, _CH)], sem1)
    cp0.wait()
    accs = _accum_edge(buf0, _CH, accs)
    cp2 = pltpu.async_copy(edge_hbm.at[pl.ds(t0 + 2 * _CH, _CH)],
                           buf0.at[pl.ds(0, _CH)], sem0)
    cp1.wait()
    accs = _accum_edge(buf1, _CH, accs)
    cp3 = pltpu.async_copy(edge_hbm.at[pl.ds(t0 + 3 * _CH, _CH)],
                           buf1.at[pl.ds(0, _CH)], sem1)
    cp2.wait()
    accs = _accum_edge(buf0, _CH, accs)
    cp3.wait()
    accs = _accum_edge(buf1, _CH, accs)

    for r in range(8):
        vout[pl.ds(r * 16, 16)] = accs[r]

    @pl.when(s < 4)
    def _edge_extra():
        pltpu.sync_copy(edge_hbm.at[pl.ds(t0 + 4 * _CH, 1)], bufx)
        for r in range(8):
            a = vout[pl.ds(r * 16, 16)]
            for k in range(8):
                a = a + bufx[0, r, pl.ds(k * 16, 16)]
            vout[pl.ds(r * 16, 16)] = a

    pltpu.sync_copy(vout, oe_hbm.at[w])

    nbase = w * 39 + jnp.minimum(w, 2)

    def _node_part(nrows):
        pltpu.sync_copy(node_hbm.at[pl.ds(nbase, nrows)],
                        buf0.at[pl.ds(0, nrows)])
        naccs = tuple(jnp.zeros((16,), jnp.float32) for _ in range(8))
        naccs = _accum_node(buf0, nrows, naccs)
        for k in range(8):
            vout[pl.ds(k * 16, 16)] = naccs[k]
        pltpu.sync_copy(vout, on_hbm.at[w])

    @pl.when(w < 2)
    def _node_a():
        _node_part(40)

    @pl.when(w >= 2)
    def _node_b():
        _node_part(39)


def _combine_body(oe_ref, on_ref, g_ref, w1_ref, b1_ref, w2_ref, b2_ref,
                  out_ref):
    oe = oe_ref[...]
    on = on_ref[...]
    it2 = lax.broadcasted_iota(jnp.int32, (2, 32), 0)
    iw2 = lax.broadcasted_iota(jnp.int32, (2, 32), 1)
    gmat = (iw2 // 16 == it2).astype(jnp.float32)
    s2 = jnp.dot(gmat, oe, preferred_element_type=jnp.float32)
    icc = lax.broadcasted_iota(jnp.int32, (128, 16), 0)
    iff = lax.broadcasted_iota(jnp.int32, (128, 16), 1)
    q0 = (iff == icc // 16).astype(jnp.float32)
    q1 = (iff == 8 + icc // 16).astype(jnp.float32)
    esum = (
        jnp.dot(s2[0:1, :], q0, preferred_element_type=jnp.float32)
        + jnp.dot(s2[1:2, :], q1, preferred_element_type=jnp.float32)
    )
    agg_e = esum * (1.0 / N_EDGES)
    ones32 = jnp.full((1, 32), 1.0, jnp.float32)
    agg_n = jnp.dot(ones32, on,
                    preferred_element_type=jnp.float32) * (1.0 / N_NODES)
    g = g_ref[...]
    w1 = w1_ref[...]
    pre = (
        jnp.dot(g, w1[0:D_GLOBAL, :], preferred_element_type=jnp.float32)
        + jnp.dot(agg_e, w1[D_GLOBAL:D_GLOBAL + D_EDGE, :],
                  preferred_element_type=jnp.float32)
        + jnp.dot(agg_n, w1[D_GLOBAL + D_EDGE:, :],
                  preferred_element_type=jnp.float32)
        + b1_ref[...]
    )
    h = jnp.maximum(pre, 0.0)
    out_ref[...] = (
        jnp.dot(h, w2_ref[...], preferred_element_type=jnp.float32)
        + b2_ref[...]
    )


def _combine(oe, on, global_attr, W1, b1_2d, W2, b2_2d):
    return pl.pallas_call(
        _combine_body,
        out_shape=jax.ShapeDtypeStruct((1, D_FEAT), jnp.float32),
    )(oe, on, global_attr, W1, b1_2d, W2, b2_2d)


def kernel(node_attr, edge_index, edge_attr, global_attr, W1, b1, W2, b2):
    del edge_index
    edge5 = (
        edge_attr.T.reshape(2, 8, 2500, 128)
        .transpose(0, 2, 1, 3)
        .reshape(5000, 8, 128)
    )
    node3 = node_attr.reshape(1250, 8, 128)
    oe, on = _sc_sums(edge5, node3)
    return _combine(oe, on, global_attr, W1, b1.reshape(1, -1), W2,
                    b2.reshape(1, -1))


# speedup vs baseline: 2.9624x; 2.9624x over previous
import jax
import jax.numpy as jnp
from jax.experimental import pallas as pl
from jax.experimental.pallas import tpu as pltpu

N_NODES = 10000
N_EDGES = 320000
D_FEAT = 128
D_EDGE = 16
D_GLOBAL = 128

NUM_BLOCKS = 10
BE = N_EDGES // (2 * NUM_BLOCKS)
BN = N_NODES // NUM_BLOCKS


def _body(edge_a_ref, edge_b_ref, node_ref, global_ref, w1_ref, b1_ref,
          w2_ref, b2_ref, out_ref, acc_e_ref, acc_n_ref):
    i = pl.program_id(0)

    @pl.when(i == 0)
    def _init():
        acc_e_ref[...] = jnp.zeros_like(acc_e_ref)
        acc_n_ref[...] = jnp.zeros_like(acc_n_ref)

    ea = edge_a_ref[...]
    eb = edge_b_ref[...]
    acc = acc_e_ref[...]
    for k in range(BE // 128):
        acc = acc + ea[:, k * 128:(k + 1) * 128]
        acc = acc + eb[:, k * 128:(k + 1) * 128]
    acc_e_ref[...] = acc
    acc_n_ref[...] += jnp.sum(node_ref[...], axis=0, keepdims=True)

    @pl.when(i == NUM_BLOCKS - 1)
    def _finish():
        esum = jnp.sum(acc_e_ref[...], axis=1, keepdims=True)
        agg_n = acc_n_ref[...] * (1.0 / N_NODES)
        g = global_ref[...]
        w1 = w1_ref[...]
        h_e = jax.lax.dot_general(
            esum * (1.0 / N_EDGES), w1[D_GLOBAL:D_GLOBAL + D_EDGE, :],
            (((0,), (0,)), ((), ())),
            preferred_element_type=jnp.float32,
        )
        pre = (
            jnp.dot(g, w1[0:D_GLOBAL, :], preferred_element_type=jnp.float32)
            + h_e
            + jnp.dot(agg_n, w1[D_GLOBAL + D_EDGE:, :],
                      preferred_element_type=jnp.float32)
            + b1_ref[...]
        )
        h = jnp.maximum(pre, 0.0)
        out_ref[...] = (
            jnp.dot(h, w2_ref[...], preferred_element_type=jnp.float32)
            + b2_ref[...]
        )


def kernel(node_attr, edge_index, edge_attr, global_attr, W1, b1, W2, b2):
    del edge_index
    b1_2d = b1.reshape(1, -1)
    b2_2d = b2.reshape(1, -1)
    edge_t = edge_attr.T
    return pl.pallas_call(
        _body,
        grid=(NUM_BLOCKS,),
        in_specs=[
            pl.BlockSpec((D_EDGE, BE), lambda i: (0, i)),
            pl.BlockSpec((D_EDGE, BE), lambda i: (0, i + NUM_BLOCKS)),
            pl.BlockSpec((BN, D_FEAT), lambda i: (i, 0)),
            pl.BlockSpec((1, D_GLOBAL), lambda i: (0, 0)),
            pl.BlockSpec((D_GLOBAL + D_EDGE + D_FEAT, 32), lambda i: (0, 0)),
            pl.BlockSpec((1, 32), lambda i: (0, 0)),
            pl.BlockSpec((32, D_FEAT), lambda i: (0, 0)),
            pl.BlockSpec((1, D_FEAT), lambda i: (0, 0)),
        ],
        out_specs=pl.BlockSpec((1, D_FEAT), lambda i: (0, 0)),
        out_shape=jax.ShapeDtypeStruct((1, D_FEAT), jnp.float32),
        scratch_shapes=[
            pltpu.VMEM((D_EDGE, 128), jnp.float32),
            pltpu.VMEM((1, D_FEAT), jnp.float32),
        ],
    )(edge_t, edge_t, node_attr, global_attr, W1, b1_2d, W2, b2_2d)


# speedup vs baseline: 3.4446x; 1.1628x over previous
import jax
import jax.numpy as jnp
from jax.experimental import pallas as pl
from jax.experimental.pallas import tpu as pltpu

N_NODES = 10000
N_EDGES = 320000
D_FEAT = 128
D_EDGE = 16
D_GLOBAL = 128

NUM_BLOCKS = 5
E_STREAMS = 4
N_STREAMS = 2
BE = N_EDGES // (E_STREAMS * NUM_BLOCKS)
BN = N_NODES // (N_STREAMS * NUM_BLOCKS)


def _body(edge_a_ref, edge_b_ref, edge_c_ref, edge_d_ref, node_a_ref,
          node_b_ref, global_ref, w1_ref, b1_ref, w2_ref, b2_ref,
          out_ref, acc_e_ref, acc_n_ref):
    i = pl.program_id(0)

    @pl.when(i == 0)
    def _init():
        acc_e_ref[...] = jnp.zeros_like(acc_e_ref)
        acc_n_ref[...] = jnp.zeros_like(acc_n_ref)

    acc = acc_e_ref[...]
    for e_ref in (edge_a_ref, edge_b_ref, edge_c_ref, edge_d_ref):
        e = e_ref[...]
        for k in range(BE // 128):
            acc = acc + e[:, k * 128:(k + 1) * 128]
    acc_e_ref[...] = acc
    acc_n_ref[...] += (
        jnp.sum(node_a_ref[...], axis=0, keepdims=True)
        + jnp.sum(node_b_ref[...], axis=0, keepdims=True)
    )

    @pl.when(i == NUM_BLOCKS - 1)
    def _finish():
        esum = jnp.sum(acc_e_ref[...], axis=1, keepdims=True)
        agg_n = acc_n_ref[...] * (1.0 / N_NODES)
        g = global_ref[...]
        w1 = w1_ref[...]
        h_e = jax.lax.dot_general(
            esum * (1.0 / N_EDGES), w1[D_GLOBAL:D_GLOBAL + D_EDGE, :],
            (((0,), (0,)), ((), ())),
            preferred_element_type=jnp.float32,
        )
        pre = (
            jnp.dot(g, w1[0:D_GLOBAL, :], preferred_element_type=jnp.float32)
            + h_e
            + jnp.dot(agg_n, w1[D_GLOBAL + D_EDGE:, :],
                      preferred_element_type=jnp.float32)
            + b1_ref[...]
        )
        h = jnp.maximum(pre, 0.0)
        out_ref[...] = (
            jnp.dot(h, w2_ref[...], preferred_element_type=jnp.float32)
            + b2_ref[...]
        )


def kernel(node_attr, edge_index, edge_attr, global_attr, W1, b1, W2, b2):
    del edge_index
    b1_2d = b1.reshape(1, -1)
    b2_2d = b2.reshape(1, -1)
    edge_t = edge_attr.T
    return pl.pallas_call(
        _body,
        grid=(NUM_BLOCKS,),
        in_specs=[
            pl.BlockSpec((D_EDGE, BE), lambda i: (0, i)),
            pl.BlockSpec((D_EDGE, BE), lambda i: (0, i + NUM_BLOCKS)),
            pl.BlockSpec((D_EDGE, BE), lambda i: (0, i + 2 * NUM_BLOCKS)),
            pl.BlockSpec((D_EDGE, BE), lambda i: (0, i + 3 * NUM_BLOCKS)),
            pl.BlockSpec((BN, D_FEAT), lambda i: (i, 0)),
            pl.BlockSpec((BN, D_FEAT), lambda i: (i + NUM_BLOCKS, 0)),
            pl.BlockSpec((1, D_GLOBAL), lambda i: (0, 0)),
            pl.BlockSpec((D_GLOBAL + D_EDGE + D_FEAT, 32), lambda i: (0, 0)),
            pl.BlockSpec((1, 32), lambda i: (0, 0)),
            pl.BlockSpec((32, D_FEAT), lambda i: (0, 0)),
            pl.BlockSpec((1, D_FEAT), lambda i: (0, 0)),
        ],
        out_specs=pl.BlockSpec((1, D_FEAT), lambda i: (0, 0)),
        out_shape=jax.ShapeDtypeStruct((1, D_FEAT), jnp.float32),
        scratch_shapes=[
            pltpu.VMEM((D_EDGE, 128), jnp.float32),
            pltpu.VMEM((1, D_FEAT), jnp.float32),
        ],
    )(edge_t, edge_t, edge_t, edge_t, node_attr, node_attr, global_attr,
      W1, b1_2d, W2, b2_2d)


# speedup vs baseline: 3.4595x; 1.0043x over previous
import jax
import jax.numpy as jnp
from jax.experimental import pallas as pl
from jax.experimental.pallas import tpu as pltpu

N_NODES = 10000
N_EDGES = 320000
D_FEAT = 128
D_EDGE = 16
D_GLOBAL = 128

NUM_BLOCKS = 5
E_STREAMS = 10
N_STREAMS = 5
BE = N_EDGES // (E_STREAMS * NUM_BLOCKS)
BN = N_NODES // (N_STREAMS * NUM_BLOCKS)


def _body(*refs):
    (edge_refs, node_refs, (global_ref, w1_ref, b1_ref, w2_ref, b2_ref),
     (out_ref,), (acc_e_ref, acc_n_ref)) = (
        refs[0:E_STREAMS], refs[E_STREAMS:E_STREAMS + N_STREAMS],
        refs[E_STREAMS + N_STREAMS:E_STREAMS + N_STREAMS + 5],
        refs[E_STREAMS + N_STREAMS + 5:E_STREAMS + N_STREAMS + 6],
        refs[E_STREAMS + N_STREAMS + 6:])
    i = pl.program_id(0)

    @pl.when(i == 0)
    def _init():
        acc_e_ref[...] = jnp.zeros_like(acc_e_ref)
        acc_n_ref[...] = jnp.zeros_like(acc_n_ref)

    acc = acc_e_ref[...]
    for e_ref in edge_refs:
        e = e_ref[...]
        for k in range(BE // 128):
            acc = acc + e[:, k * 128:(k + 1) * 128]
    acc_e_ref[...] = acc
    nsum = jnp.sum(node_refs[0][...], axis=0, keepdims=True)
    for n_ref in node_refs[1:]:
        nsum = nsum + jnp.sum(n_ref[...], axis=0, keepdims=True)
    acc_n_ref[...] += nsum

    @pl.when(i == NUM_BLOCKS - 1)
    def _finish():
        esum = jnp.sum(acc_e_ref[...], axis=1, keepdims=True)
        agg_n = acc_n_ref[...] * (1.0 / N_NODES)
        g = global_ref[...]
        w1 = w1_ref[...]
        h_e = jax.lax.dot_general(
            esum * (1.0 / N_EDGES), w1[D_GLOBAL:D_GLOBAL + D_EDGE, :],
            (((0,), (0,)), ((), ())),
            preferred_element_type=jnp.float32,
        )
        pre = (
            jnp.dot(g, w1[0:D_GLOBAL, :], preferred_element_type=jnp.float32)
            + h_e
            + jnp.dot(agg_n, w1[D_GLOBAL + D_EDGE:, :],
                      preferred_element_type=jnp.float32)
            + b1_ref[...]
        )
        h = jnp.maximum(pre, 0.0)
        out_ref[...] = (
            jnp.dot(h, w2_ref[...], preferred_element_type=jnp.float32)
            + b2_ref[...]
        )


def kernel(node_attr, edge_index, edge_attr, global_attr, W1, b1, W2, b2):
    del edge_index
    b1_2d = b1.reshape(1, -1)
    b2_2d = b2.reshape(1, -1)
    edge_t = edge_attr.T
    return pl.pallas_call(
        _body,
        grid=(NUM_BLOCKS,),
        in_specs=[
            *[pl.BlockSpec((D_EDGE, BE),
                           (lambda j: lambda i: (0, i + j * NUM_BLOCKS))(j))
              for j in range(E_STREAMS)],
            *[pl.BlockSpec((BN, D_FEAT),
                           (lambda j: lambda i: (i + j * NUM_BLOCKS, 0))(j))
              for j in range(N_STREAMS)],
            pl.BlockSpec((1, D_GLOBAL), lambda i: (0, 0)),
            pl.BlockSpec((D_GLOBAL + D_EDGE + D_FEAT, 32), lambda i: (0, 0)),
            pl.BlockSpec((1, 32), lambda i: (0, 0)),
            pl.BlockSpec((32, D_FEAT), lambda i: (0, 0)),
            pl.BlockSpec((1, D_FEAT), lambda i: (0, 0)),
        ],
        out_specs=pl.BlockSpec((1, D_FEAT), lambda i: (0, 0)),
        out_shape=jax.ShapeDtypeStruct((1, D_FEAT), jnp.float32),
        scratch_shapes=[
            pltpu.VMEM((D_EDGE, 128), jnp.float32),
            pltpu.VMEM((1, D_FEAT), jnp.float32),
        ],
    )(*([edge_t] * E_STREAMS), *([node_attr] * N_STREAMS), global_attr,
      W1, b1_2d, W2, b2_2d)
